# baseline (speedup 1.0000x reference)
import functools

import jax
import jax.numpy as jnp
from jax import lax
from jax.experimental import pallas as pl
from jax.experimental.pallas import tpu as pltpu
from jax.experimental.pallas import tpu_sc as plsc

ALPHA = 0.025
NC = 2
NS = 16
NW = NC * NS
LANES = 16

N = 1000000
D = 64
B = 16384
K = 5
T = K + 1

EPW = B // NW
CH = 128
NCH = EPW // CH
ROWS_PW = N // NW

_mesh = plsc.VectorSubcoreMesh(core_axis_name="c", subcore_axis_name="s")
_sc_params = pltpu.CompilerParams(
    use_tc_tiling_on_sc=False, needs_layout_passes=False)


def _wid():
    return lax.axis_index("s") * NC + lax.axis_index("c")


@functools.partial(
    pl.kernel,
    out_type=(
        jax.ShapeDtypeStruct((B, D), jnp.float32),
        jax.ShapeDtypeStruct((T, B, D), jnp.float32),
    ),
    mesh=_mesh,
    compiler_params=_sc_params,
    scratch_types=[
        pltpu.VMEM((CH,), jnp.int32),
        pltpu.VMEM((T, CH), jnp.int32),
        pltpu.VMEM((CH * K,), jnp.int32),
        pltpu.VMEM((CH, D), jnp.float32),
        pltpu.VMEM((T, CH, D), jnp.float32),
        pltpu.SemaphoreType.DMA,
        pltpu.SemaphoreType.DMA,
    ],
)
def _k1_gather(vertex, context, u, v, negf, vecu_out, vecv_out,
               uv, tgtv, negv, urows, crows, sem_u, sem_c):
    wid = _wid()
    lane = lax.iota(jnp.int32, LANES)
    for c in range(NCH):
        base = (wid * NCH + c) * CH
        pltpu.sync_copy(u.at[pl.ds(base, CH)], uv)
        pltpu.sync_copy(v.at[pl.ds(base, CH)], tgtv.at[0])
        pltpu.sync_copy(negf.at[pl.ds(base * K, CH * K)], negv)
        cp_u = pltpu.async_copy(vertex.at[uv], urows, sem_u)
        for t in range(1, T):
            for s in range(CH // LANES):
                idx = (lane + s * LANES) * K + (t - 1)
                col = plsc.load_gather(negv, [idx])
                tgtv[t, pl.ds(s * LANES, LANES)] = col
        cps = []
        for t in range(T):
            cps.append(
                pltpu.async_copy(context.at[tgtv.at[t]], crows.at[t], sem_c))
        cp_u.wait()
        pltpu.sync_copy(urows, vecu_out.at[pl.ds(base, CH)])
        for t in range(T):
            cps[t].wait()
            pltpu.sync_copy(crows.at[t], vecv_out.at[t, pl.ds(base, CH)])


BLK = 2048


def _k2_body(vecu_ref, vecv_ref, verr_ref):
    vu = vecu_ref[...]
    vv = vecv_ref[...]
    dots = jnp.sum(vv * vu[None, :, :], axis=-1)
    f = 1.0 / (1.0 + jnp.exp(-dots))
    t_idx = lax.broadcasted_iota(jnp.int32, (T, BLK), 0)
    label = jnp.where(t_idx == 0, 1.0, 0.0)
    g = ALPHA * (label - f)
    verr_ref[...] = jnp.sum(g[:, :, None] * vv, axis=0)


def _k2_dense(vecu, vecv):
    return pl.pallas_call(
        _k2_body,
        grid=(B // BLK,),
        in_specs=[
            pl.BlockSpec((BLK, D), lambda i: (i, 0)),
            pl.BlockSpec((T, BLK, D), lambda i: (0, i, 0)),
        ],
        out_specs=pl.BlockSpec((BLK, D), lambda i: (i, 0)),
        out_shape=jax.ShapeDtypeStruct((B, D), jnp.float32),
    )(vecu, vecv)


@functools.partial(
    pl.kernel,
    out_type=(),
    mesh=_mesh,
    compiler_params=_sc_params,
    scratch_types=[
        pltpu.VMEM((B,), jnp.int32),
        pltpu.VMEM((B + LANES,), jnp.int32),
        pltpu.VMEM((B + LANES,), jnp.int32),
        pltpu.VMEM((LANES, D), jnp.float32),
        pltpu.VMEM((LANES, D), jnp.float32),
        pltpu.VMEM((LANES,), jnp.int32),
        pltpu.VMEM((LANES,), jnp.float32),
        pltpu.SemaphoreType.DMA,
        pltpu.SemaphoreType.DMA,
    ],
)
def _k3_scatter(out_ref, u, verr, u_all, my_eid, my_row, vbuf, obuf,
                rowscr, valscr, sem_a, sem_b):
    wid = _wid()
    lo = wid * ROWS_PW
    hi = lo + ROWS_PW
    lane = lax.iota(jnp.int32, LANES)
    perms = [(lane + j) % LANES for j in range(LANES)]

    pltpu.sync_copy(u, u_all)

    def scan_body(i, off):
        uvec = u_all[pl.ds(i * LANES, LANES)]
        msk = (uvec >= lo) & (uvec < hi)
        eids = lane + i * LANES
        plsc.store_compressed(my_eid.at[pl.ds(off, LANES)], eids, mask=msk)
        plsc.store_compressed(my_row.at[pl.ds(off, LANES)], uvec, mask=msk)
        return off + jnp.sum(msk.astype(jnp.int32))

    m = lax.fori_loop(0, B // LANES, scan_body, 0)
    my_row[pl.ds(m, LANES)] = jnp.full((LANES,), lo, jnp.int32)
    my_eid[pl.ds(m, LANES)] = jnp.zeros((LANES,), jnp.int32)
    nwaves = (m + LANES - 1) // LANES

    def wave(w, carry):
        r = my_row[pl.ds(w * LANES, LANES)]
        e = my_eid[pl.ds(w * LANES, LANES)]
        cp_v = pltpu.async_copy(verr.at[e], vbuf, sem_a)
        cp_o = pltpu.async_copy(out_ref.at[r], obuf, sem_b)
        cp_v.wait()
        cp_o.wait()

        @pl.when(w * LANES + LANES > m)
        def _():
            for l in range(LANES):
                s = jnp.where(w * LANES + l < m, 1.0, 0.0)
                for q in range(D // LANES):
                    sl = pl.ds(q * LANES, LANES)
                    vbuf[l, sl] = vbuf[l, sl] * s

        rowscr[...] = r
        anyd = jnp.asarray(False)
        for j in range(1, LANES):
            rj = plsc.load_gather(rowscr, [perms[j]])
            anyd = anyd | jnp.any(r == rj)

        @pl.when(jnp.logical_not(anyd))
        def _():
            for l in range(LANES):
                for q in range(D // LANES):
                    sl = pl.ds(q * LANES, LANES)
                    obuf[l, sl] = obuf[l, sl] + vbuf[l, sl]

        @pl.when(anyd)
        def _():
            for d in range(D):
                dsp = jnp.full((LANES,), d, jnp.int32)
                vd = plsc.load_gather(vbuf, [lane, dsp])
                valscr[...] = vd
                acc = plsc.load_gather(obuf, [lane, dsp])
                for j in range(LANES):
                    rj = plsc.load_gather(rowscr, [perms[j]])
                    vj = plsc.load_gather(valscr, [perms[j]])
                    acc = acc + jnp.where(r == rj, vj, 0.0)
                plsc.store_scatter(obuf, [lane, dsp], acc)

        pltpu.async_copy(obuf, out_ref.at[r], sem_b).wait()
        return carry

    lax.fori_loop(0, nwaves, wave, 0)


def kernel(emb_vertex, emb_context, u, v, neg):
    u = u.astype(jnp.int32)
    v = v.astype(jnp.int32)
    negf = neg.astype(jnp.int32).reshape(-1)
    vecu, vecv = _k1_gather(emb_vertex, emb_context, u, v, negf)
    verr = _k2_dense(vecu, vecv)
    out = jax.new_ref(emb_vertex)
    _k3_scatter(out, u, verr)
    return out[...]


# speedup vs baseline: 1.4825x; 1.4825x over previous
import functools

import jax
import jax.numpy as jnp
from jax import lax
from jax.experimental import pallas as pl
from jax.experimental.pallas import tpu as pltpu
from jax.experimental.pallas import tpu_sc as plsc

ALPHA = 0.025
NC = 2
NS = 16
NW = NC * NS
LANES = 16

N = 1000000
D = 64
B = 16384
K = 5
T = K + 1

EPW = B // NW
NCH = EPW // LANES
ROWS_PW = N // NW

_mesh = plsc.VectorSubcoreMesh(core_axis_name="c", subcore_axis_name="s")
_sc_params = pltpu.CompilerParams(
    use_tc_tiling_on_sc=True, needs_layout_passes=False)

_LANE = None


def _wid():
    return lax.axis_index("s") * NC + lax.axis_index("c")


def _ext(vec, l, lane):
    return jnp.sum(jnp.where(lane == l, vec, jnp.zeros_like(vec)))


@functools.partial(
    pl.kernel,
    out_type=(
        jax.ShapeDtypeStruct((B, D), jnp.float32),
        jax.ShapeDtypeStruct((T, B, D), jnp.float32),
    ),
    mesh=_mesh,
    compiler_params=_sc_params,
    scratch_types=[
        pltpu.VMEM((LANES,), jnp.int32),
        pltpu.VMEM((LANES,), jnp.int32),
        pltpu.VMEM((LANES * K,), jnp.int32),
        pltpu.VMEM((LANES, D), jnp.float32),
        pltpu.VMEM((T, LANES, D), jnp.float32),
        pltpu.SemaphoreType.DMA,
        pltpu.SemaphoreType.DMA,
        pltpu.SemaphoreType.DMA,
    ],
)
def _k1_gather(vertex, context, u, v, negf, vecu_out, vecv_out,
               uv, vv, negv, urows, crows, sem_i, sem_g, sem_w):
    wid = _wid()
    lane = lax.iota(jnp.int32, LANES)

    def chunk_body(c, carry):
        base = wid * EPW + c * LANES
        pltpu.sync_copy(u.at[pl.ds(base, LANES)], uv)
        pltpu.sync_copy(v.at[pl.ds(base, LANES)], vv)
        pltpu.sync_copy(negf.at[pl.ds(base * K, LANES * K)], negv)
        uvec = uv[...]
        vvec = vv[...]
        nblk = [negv[pl.ds(b * LANES, LANES)] for b in range(LANES * K // LANES)]
        fired = []
        for l in range(LANES):
            ul = _ext(uvec, l, lane)
            fired.append(pltpu.async_copy(
                vertex.at[pl.ds(ul, 1), :], urows.at[pl.ds(l, 1), :], sem_g))
            vl = _ext(vvec, l, lane)
            fired.append(pltpu.async_copy(
                context.at[pl.ds(vl, 1), :], crows.at[0, pl.ds(l, 1), :],
                sem_g))
        for t in range(1, T):
            for l in range(LANES):
                fidx = l * K + (t - 1)
                nl = _ext(nblk[fidx // LANES], fidx % LANES, lane)
                fired.append(pltpu.async_copy(
                    context.at[pl.ds(nl, 1), :], crows.at[t, pl.ds(l, 1), :],
                    sem_g))
        for cp in fired:
            cp.wait()
        wr = [pltpu.async_copy(urows, vecu_out.at[pl.ds(base, LANES)], sem_w)]
        for t in range(T):
            wr.append(pltpu.async_copy(
                crows.at[t], vecv_out.at[t, pl.ds(base, LANES)], sem_w))
        for cp in wr:
            cp.wait()
        return carry

    lax.fori_loop(0, NCH, chunk_body, 0)


BLK = 2048


def _k2_body(vecu_ref, vecv_ref, verr_ref):
    vu = vecu_ref[...]
    vv = vecv_ref[...]
    dots = jnp.sum(vv * vu[None, :, :], axis=-1)
    f = 1.0 / (1.0 + jnp.exp(-dots))
    t_idx = lax.broadcasted_iota(jnp.int32, (T, BLK), 0)
    label = jnp.where(t_idx == 0, 1.0, 0.0)
    g = ALPHA * (label - f)
    verr_ref[...] = jnp.sum(g[:, :, None] * vv, axis=0)


def _k2_dense(vecu, vecv):
    return pl.pallas_call(
        _k2_body,
        grid=(B // BLK,),
        in_specs=[
            pl.BlockSpec((BLK, D), lambda i: (i, 0)),
            pl.BlockSpec((T, BLK, D), lambda i: (0, i, 0)),
        ],
        out_specs=pl.BlockSpec((BLK, D), lambda i: (i, 0)),
        out_shape=jax.ShapeDtypeStruct((B, D), jnp.float32),
    )(vecu, vecv)


@functools.partial(
    pl.kernel,
    out_type=(),
    mesh=_mesh,
    compiler_params=_sc_params,
    scratch_types=[
        pltpu.VMEM((B,), jnp.int32),
        pltpu.VMEM((B + LANES,), jnp.int32),
        pltpu.VMEM((B + LANES,), jnp.int32),
        pltpu.VMEM((LANES, D), jnp.float32),
        pltpu.VMEM((LANES, D), jnp.float32),
        pltpu.VMEM((LANES,), jnp.int32),
        pltpu.VMEM((LANES,), jnp.float32),
        pltpu.SemaphoreType.DMA,
        pltpu.SemaphoreType.DMA,
        pltpu.SemaphoreType.DMA,
    ],
)
def _k3_scatter(out_ref, u, verr, u_all, my_eid, my_row, vbuf, obuf,
                rowscr, valscr, sem_v, sem_o, sem_w):
    wid = _wid()
    lo = wid * ROWS_PW
    hi = lo + ROWS_PW
    lane = lax.iota(jnp.int32, LANES)
    perms_f = [(lane + j) % LANES for j in range(LANES)]
    perms_b = [(lane - j) % LANES for j in range(LANES)]

    pltpu.sync_copy(u, u_all)

    def scan_body(i, off):
        uvec = u_all[pl.ds(i * LANES, LANES)]
        msk = (uvec >= lo) & (uvec < hi)
        eids = lane + i * LANES
        plsc.store_compressed(my_eid.at[pl.ds(off, LANES)], eids, mask=msk)
        plsc.store_compressed(my_row.at[pl.ds(off, LANES)], uvec, mask=msk)
        return off + jnp.sum(msk.astype(jnp.int32))

    m = lax.fori_loop(0, B // LANES, scan_body, 0)
    nwaves = (m + LANES - 1) // LANES

    def wave(w, carry):
        r = my_row[pl.ds(w * LANES, LANES)]
        e = my_eid[pl.ds(w * LANES, LANES)]
        validv = lane < (m - w * LANES)
        rmask = jnp.where(validv, r, -1 - lane)

        for l in range(LANES):
            el = _ext(e, l, lane)
            @pl.when(w * LANES + l < m)
            def _(el=el, l=l):
                pltpu.async_copy(
                    verr.at[pl.ds(el, 1), :], vbuf.at[pl.ds(l, 1), :],
                    sem_v)
        rowscr[...] = rmask
        prev_eq = lane < 0
        for j in range(1, LANES):
            rj = plsc.load_gather(rowscr, [perms_b[j]])
            prev_eq = prev_eq | ((rmask == rj) & (lane >= j))
        firstv = jnp.logical_not(prev_eq)
        anyd = jnp.any(prev_eq)
        f01 = jnp.where(firstv, 1, 0)

        for l in range(LANES):
            @pl.when(w * LANES + l < m)
            def _(l=l):
                pltpu.make_async_copy(
                    verr.at[pl.ds(0, 1), :], vbuf.at[pl.ds(l, 1), :],
                    sem_v).wait()

        @pl.when(anyd)
        def _():
            vld01 = jnp.where(validv, 1.0, 0.0)
            fst01 = jnp.where(firstv, 1.0, 0.0)

            def comb_body(d, carry2):
                dsp = jnp.full((LANES,), d, jnp.int32)
                vd = plsc.load_gather(vbuf, [lane, dsp]) * vld01
                valscr[...] = vd
                acc = jnp.zeros((LANES,), jnp.float32)
                for j in range(LANES):
                    rj = plsc.load_gather(rowscr, [perms_f[j]])
                    vj = plsc.load_gather(valscr, [perms_f[j]])
                    acc = acc + jnp.where(rmask == rj, vj, 0.0)
                plsc.store_scatter(vbuf, [lane, dsp], acc * fst01)
                return carry2

            lax.fori_loop(0, D, comb_body, 0)

        rvec = rmask
        for l in range(LANES):
            rl = _ext(rvec, l, lane)
            fl = _ext(f01, l, lane)
            @pl.when((w * LANES + l < m) & (fl == 1))
            def _(rl=rl, l=l):
                pltpu.async_copy(
                    out_ref.at[pl.ds(rl, 1), :], obuf.at[pl.ds(l, 1), :],
                    sem_o)
        for l in range(LANES):
            fl = _ext(f01, l, lane)
            @pl.when((w * LANES + l < m) & (fl == 1))
            def _(l=l):
                pltpu.make_async_copy(
                    out_ref.at[pl.ds(0, 1), :], obuf.at[pl.ds(l, 1), :],
                    sem_o).wait()

        for l in range(LANES):
            for q in range(D // LANES):
                sl = pl.ds(q * LANES, LANES)
                obuf[l, sl] = obuf[l, sl] + vbuf[l, sl]

        for l in range(LANES):
            rl = _ext(rvec, l, lane)
            fl = _ext(f01, l, lane)
            @pl.when((w * LANES + l < m) & (fl == 1))
            def _(rl=rl, l=l):
                pltpu.async_copy(
                    obuf.at[pl.ds(l, 1), :], out_ref.at[pl.ds(rl, 1), :],
                    sem_w)
        for l in range(LANES):
            fl = _ext(f01, l, lane)
            @pl.when((w * LANES + l < m) & (fl == 1))
            def _(l=l):
                pltpu.make_async_copy(
                    obuf.at[pl.ds(l, 1), :], out_ref.at[pl.ds(0, 1), :],
                    sem_w).wait()
        return carry

    lax.fori_loop(0, nwaves, wave, 0)


def kernel(emb_vertex, emb_context, u, v, neg):
    u = u.astype(jnp.int32)
    v = v.astype(jnp.int32)
    negf = neg.astype(jnp.int32).reshape(-1)
    vecu, vecv = _k1_gather(emb_vertex, emb_context, u, v, negf)
    verr = _k2_dense(vecu, vecv)
    out = jax.new_ref(emb_vertex)
    _k3_scatter(out, u, verr)
    return out[...]
